# baseline (speedup 1.0000x reference)
import functools

import jax
import jax.numpy as jnp
from jax import lax
from jax.experimental import pallas as pl
from jax.experimental.pallas import tpu as pltpu
from jax.experimental.pallas import tpu_sc as plsc

NUM_ROWS = 1_000_000
DIM = 64
BATCH = 16384
HIST = 200
HALF = HIST // 2
PADH = 104
BUF = 2 * PADH
NVR = BUF // 16

NC = 2
NS = 16
NW = NC * NS
BPW = BATCH // NW
CHUNK = 256
NCHUNKS = BPW // CHUNK
NBUF = 16
NGROUPS = CHUNK // NBUF


RB = 8000
NBLK = NUM_ROWS // RB
OB = 8192
PTOT = NBLK * OB


def _tc_heads(table1d, w4):
    hb = RB // 2

    def body(t_ref, w_ref, o1_ref, o2_ref):
        t = t_ref[...].reshape(hb, 2 * DIM)
        r = lax.dot_general(w_ref[...], t, (((1,), (1,)), ((), ())),
                            preferred_element_type=jnp.float32)
        z = jnp.zeros((OB // 2 - hb,), jnp.float32)
        o1_ref[...] = jnp.concatenate([r[0], z, r[1], z])
        o2_ref[...] = jnp.concatenate([r[2], z, r[3], z])

    out1d = jax.ShapeDtypeStruct((PTOT,), jnp.float32)
    return pl.pallas_call(
        body,
        grid=(NBLK,),
        in_specs=[
            pl.BlockSpec((RB * DIM,), lambda i: (i,)),
            pl.BlockSpec((4, 2 * DIM), lambda i: (0, 0)),
        ],
        out_specs=[pl.BlockSpec((OB,), lambda i: (i,))] * 2,
        out_shape=[out1d] * 2,
    )(table1d, w4)


def _sc_gather_reduce(x3, p1, p2, bias_vec):
    mesh = plsc.VectorSubcoreMesh(core_axis_name="c", subcore_axis_name="s",
                                  num_cores=NC, num_subcores=NS)

    @functools.partial(
        pl.kernel,
        out_type=[jax.ShapeDtypeStruct((BATCH,), jnp.float32),
                  jax.ShapeDtypeStruct((BATCH,), jnp.float32)],
        mesh=mesh,
        scratch_types=[
            pltpu.VMEM((CHUNK, 2, HALF), jnp.int32),
            pltpu.VMEM((NBUF, BUF), jnp.float32),
            pltpu.VMEM((NBUF, BUF), jnp.float32),
            pltpu.VMEM((CHUNK,), jnp.float32),
            pltpu.VMEM((CHUNK,), jnp.float32),
            pltpu.VMEM((16,), jnp.float32),
            pltpu.SemaphoreType.DMA((NBUF,)),
        ],
        compiler_params=pltpu.CompilerParams(use_tc_tiling_on_sc=False,
                                             needs_layout_passes=False),
    )
    def body(x_hbm, p1_hbm, p2_hbm, bias_hbm, out1_hbm, out2_hbm,
             idx_v, buf1_v, buf2_v, o1_v, o2_v, bias_v, sems):
        wid = lax.axis_index("s") * NC + lax.axis_index("c")
        base = wid * BPW
        pltpu.sync_copy(bias_hbm, bias_v)
        bv = bias_v[...]
        b1s = bv[0]
        b2s = bv[1]
        lanes = lax.iota(jnp.int32, 16)

        zeros16 = jnp.broadcast_to(jnp.float32(0.0), (16,))
        for s in range(NBUF):
            for j in range(NVR):
                buf1_v[s, pl.ds(16 * j, 16)] = zeros16
                buf2_v[s, pl.ds(16 * j, 16)] = zeros16

        def gathers(li, s):
            return [
                (p1_hbm.at[idx_v.at[li, 0]], buf1_v.at[s].at[pl.ds(0, HALF)]),
                (p1_hbm.at[idx_v.at[li, 1]], buf1_v.at[s].at[pl.ds(PADH, HALF)]),
                (p2_hbm.at[idx_v.at[li, 0]], buf2_v.at[s].at[pl.ds(0, HALF)]),
                (p2_hbm.at[idx_v.at[li, 1]], buf2_v.at[s].at[pl.ds(PADH, HALF)]),
            ]

        def issue(li, s):
            for src, dst in gathers(li, s):
                pltpu.async_copy(src, dst, sems.at[s])

        def drain(li, s):
            for src, dst in gathers(li, s):
                pltpu.make_async_copy(src, dst, sems.at[s]).wait()

        def reduce(s, v1, v2):
            acc1 = buf1_v[s, pl.ds(0, 16)]
            acc2 = buf2_v[s, pl.ds(0, 16)]
            for j in range(1, NVR):
                acc1 = acc1 + buf1_v[s, pl.ds(16 * j, 16)]
                acc2 = acc2 + buf2_v[s, pl.ds(16 * j, 16)]
            s1 = jnp.sum(acc1) + b1s
            s2 = jnp.sum(acc2) + b2s
            sel = lanes == s
            v1 = jnp.where(sel, jnp.broadcast_to(s1, (16,)), v1)
            v2 = jnp.where(sel, jnp.broadcast_to(s2, (16,)), v2)
            return v1, v2

        def chunk_body(ci, _):
            cbase = base + ci * CHUNK
            pltpu.sync_copy(x_hbm.at[pl.ds(cbase, CHUNK)], idx_v)
            for b in range(NBUF):
                issue(b, b)

            def group_body(g, _):
                v1 = zeros16
                v2 = zeros16
                for b in range(NBUF):
                    li = g * NBUF + b
                    drain(li, b)
                    v1, v2 = reduce(b, v1, v2)

                    @pl.when(li + NBUF < CHUNK)
                    def _():
                        issue(li + NBUF, b)
                o1_v[pl.ds(g * NBUF, 16)] = v1
                o2_v[pl.ds(g * NBUF, 16)] = v2
                return 0

            lax.fori_loop(0, NGROUPS, group_body, 0)
            pltpu.sync_copy(o1_v, out1_hbm.at[pl.ds(cbase, CHUNK)])
            pltpu.sync_copy(o2_v, out2_hbm.at[pl.ds(cbase, CHUNK)])
            return 0

        lax.fori_loop(0, NCHUNKS, chunk_body, 0)

    return body(x3, p1, p2, bias_vec)


@jax.jit
def kernel(x, table, W1, b1, W2, b2):
    s = 1.0 / HIST
    z = jnp.zeros((1, DIM), jnp.float32)
    w4 = jnp.concatenate([
        jnp.concatenate([W1 * s, z], axis=1),
        jnp.concatenate([z, W1 * s], axis=1),
        jnp.concatenate([W2 * s, z], axis=1),
        jnp.concatenate([z, W2 * s], axis=1),
    ], axis=0)
    bias_vec = jnp.concatenate(
        [b1, b2, jnp.zeros((14,), jnp.float32)])
    p1, p2 = _tc_heads(table.reshape(-1), w4)
    x = x.astype(jnp.int32)
    rem = x % RB
    pos = (x // RB) * OB + (rem & 1) * (OB // 2) + (rem >> 1)
    x3 = pos.reshape(BATCH, 2, HALF)
    out1, out2 = _sc_gather_reduce(x3, p1, p2, bias_vec)
    return (out1, out2)


# speedup vs baseline: 1.0971x; 1.0971x over previous
import functools

import jax
import jax.numpy as jnp
from jax import lax
from jax.experimental import pallas as pl
from jax.experimental.pallas import tpu as pltpu
from jax.experimental.pallas import tpu_sc as plsc

NUM_ROWS = 1_000_000
DIM = 64
BATCH = 16384
HIST = 200
HALF = HIST // 2

NC = 2
NS = 16
NW = NC * NS
BPW = BATCH // NW
CHUNK = 128
NCHUNKS = BPW // CHUNK
NBUF = 4
NGROUPS = CHUNK // NBUF


def _sc_embed_sum(x3, table):
    mesh = plsc.VectorSubcoreMesh(core_axis_name="c", subcore_axis_name="s",
                                  num_cores=NC, num_subcores=NS)

    @functools.partial(
        pl.kernel,
        out_type=jax.ShapeDtypeStruct((BATCH, DIM), jnp.float32),
        mesh=mesh,
        scratch_types=[
            pltpu.VMEM((CHUNK, 2, HALF), jnp.int32),
            pltpu.VMEM((NBUF, HIST, DIM), jnp.float32),
            pltpu.VMEM((CHUNK, DIM), jnp.float32),
            pltpu.SemaphoreType.DMA((NBUF,)),
        ],
        compiler_params=pltpu.CompilerParams(use_tc_tiling_on_sc=False,
                                             needs_layout_passes=False),
    )
    def body(x_hbm, t_hbm, out_hbm, idx_v, rows_v, hbuf_v, sems):
        wid = lax.axis_index("s") * NC + lax.axis_index("c")
        base = wid * BPW

        def gathers(li, s):
            return [
                (t_hbm.at[idx_v.at[li, 0]], rows_v.at[s].at[pl.ds(0, HALF)]),
                (t_hbm.at[idx_v.at[li, 1]], rows_v.at[s].at[pl.ds(HALF, HALF)]),
            ]

        def issue(li, s):
            for src, dst in gathers(li, s):
                pltpu.async_copy(src, dst, sems.at[s])

        def drain(li, s):
            for src, dst in gathers(li, s):
                pltpu.make_async_copy(src, dst, sems.at[s]).wait()

        def reduce(li, s):
            acc = [rows_v[s, 0, pl.ds(16 * c, 16)] for c in range(4)]

            def rstep(j, acc):
                a0, a1, a2, a3 = acc
                for u in range(4):
                    jj = 4 * j + 1 + u
                    a0 = a0 + rows_v[s, jj, pl.ds(0, 16)]
                    a1 = a1 + rows_v[s, jj, pl.ds(16, 16)]
                    a2 = a2 + rows_v[s, jj, pl.ds(32, 16)]
                    a3 = a3 + rows_v[s, jj, pl.ds(48, 16)]
                return (a0, a1, a2, a3)

            acc = lax.fori_loop(0, (HIST - 4) // 4, rstep, tuple(acc))
            a0, a1, a2, a3 = acc
            for jj in range(HIST - 3, HIST):
                a0 = a0 + rows_v[s, jj, pl.ds(0, 16)]
                a1 = a1 + rows_v[s, jj, pl.ds(16, 16)]
                a2 = a2 + rows_v[s, jj, pl.ds(32, 16)]
                a3 = a3 + rows_v[s, jj, pl.ds(48, 16)]
            hbuf_v[li, pl.ds(0, 16)] = a0
            hbuf_v[li, pl.ds(16, 16)] = a1
            hbuf_v[li, pl.ds(32, 16)] = a2
            hbuf_v[li, pl.ds(48, 16)] = a3

        def chunk_body(ci, _):
            cbase = base + ci * CHUNK
            pltpu.sync_copy(x_hbm.at[pl.ds(cbase, CHUNK)], idx_v)
            for b in range(NBUF):
                issue(b, b)

            def group_body(g, _):
                for b in range(NBUF):
                    li = g * NBUF + b
                    drain(li, b)
                    reduce(li, b)

                    @pl.when(li + NBUF < CHUNK)
                    def _():
                        issue(li + NBUF, b)
                return 0

            lax.fori_loop(0, NGROUPS, group_body, 0)
            pltpu.sync_copy(hbuf_v, out_hbm.at[pl.ds(cbase, CHUNK)])
            return 0

        lax.fori_loop(0, NCHUNKS, chunk_body, 0)

    return body(x3, table)


def _tc_heads(h, w12, bias2):
    rb = 2048

    def body(h_ref, w_ref, b_ref, o1_ref, o2_ref):
        r = lax.dot_general(w_ref[...], h_ref[...], (((1,), (1,)), ((), ())),
                            preferred_element_type=jnp.float32)
        b = b_ref[...]
        o1_ref[...] = r[0] + b[0]
        o2_ref[...] = r[1] + b[1]

    out1d = jax.ShapeDtypeStruct((BATCH,), jnp.float32)
    return pl.pallas_call(
        body,
        grid=(BATCH // rb,),
        in_specs=[
            pl.BlockSpec((rb, DIM), lambda i: (i, 0)),
            pl.BlockSpec((2, DIM), lambda i: (0, 0)),
            pl.BlockSpec((2,), lambda i: (0,)),
        ],
        out_specs=[pl.BlockSpec((rb,), lambda i: (i,))] * 2,
        out_shape=[out1d] * 2,
    )(h, w12, bias2)


@jax.jit
def kernel(x, table, W1, b1, W2, b2):
    w12 = jnp.concatenate([W1, W2], axis=0) * (1.0 / HIST)
    bias2 = jnp.concatenate([b1, b2])
    x3 = x.astype(jnp.int32).reshape(BATCH, 2, HALF)
    h = _sc_embed_sum(x3, table)
    out1, out2 = _tc_heads(h, w12, bias2)
    return (out1, out2)


# speedup vs baseline: 1.3699x; 1.2487x over previous
import functools

import jax
import jax.numpy as jnp
from jax import lax
from jax.experimental import pallas as pl
from jax.experimental.pallas import tpu as pltpu
from jax.experimental.pallas import tpu_sc as plsc

NUM_ROWS = 1_000_000
DIM = 64
BATCH = 16384
HIST = 200
HALF = HIST // 2
PADH = 104
BUF = 2 * PADH
NVR = BUF // 16

NC = 2
NS = 16
NW = NC * NS
BPW = BATCH // NW
CHUNK = 256
NCHUNKS = BPW // CHUNK
NBUF = 16
NGROUPS = CHUNK // NBUF


def _tc_heads_packed(table, w12):
    rb = 8192
    grid = pl.cdiv(NUM_ROWS, rb)

    def body(t_ref, w_ref, o_ref):
        r = lax.dot_general(w_ref[...], t_ref[...], (((1,), (1,)), ((), ())),
                            preferred_element_type=jnp.float32)
        u0 = lax.bitcast_convert_type(r[0], jnp.uint32)
        u1 = lax.bitcast_convert_type(r[1], jnp.uint32)
        r0 = (u0 + jnp.uint32(0x7FFF) + ((u0 >> 16) & jnp.uint32(1))) \
            & jnp.uint32(0xFFFF0000)
        r1 = (u1 + jnp.uint32(0x7FFF) + ((u1 >> 16) & jnp.uint32(1))) \
            & jnp.uint32(0xFFFF0000)
        q = (r0 >> 16) | r1
        o_ref[...] = lax.bitcast_convert_type(q, jnp.float32)

    return pl.pallas_call(
        body,
        grid=(grid,),
        in_specs=[
            pl.BlockSpec((rb, DIM), lambda i: (i, 0)),
            pl.BlockSpec((2, DIM), lambda i: (0, 0)),
        ],
        out_specs=pl.BlockSpec((rb,), lambda i: (i,)),
        out_shape=jax.ShapeDtypeStruct((NUM_ROWS,), jnp.float32),
    )(table, w12)


def _sc_gather_reduce(x3, q, bias_vec):
    mesh = plsc.VectorSubcoreMesh(core_axis_name="c", subcore_axis_name="s",
                                  num_cores=NC, num_subcores=NS)

    @functools.partial(
        pl.kernel,
        out_type=[jax.ShapeDtypeStruct((BATCH,), jnp.float32),
                  jax.ShapeDtypeStruct((BATCH,), jnp.float32)],
        mesh=mesh,
        scratch_types=[
            pltpu.VMEM((CHUNK, 2, HALF), jnp.int32),
            pltpu.VMEM((NBUF, BUF), jnp.float32),
            pltpu.VMEM((CHUNK,), jnp.float32),
            pltpu.VMEM((CHUNK,), jnp.float32),
            pltpu.VMEM((16,), jnp.float32),
            pltpu.SemaphoreType.DMA((NBUF,)),
        ],
        compiler_params=pltpu.CompilerParams(use_tc_tiling_on_sc=False,
                                             needs_layout_passes=False),
    )
    def body(x_hbm, q_hbm, bias_hbm, out1_hbm, out2_hbm,
             idx_v, buf_v, o1_v, o2_v, bias_v, sems):
        wid = lax.axis_index("s") * NC + lax.axis_index("c")
        base = wid * BPW
        pltpu.sync_copy(bias_hbm, bias_v)
        bv = bias_v[...]
        b1s = bv[0]
        b2s = bv[1]
        lanes = lax.iota(jnp.int32, 16)
        himask = jnp.broadcast_to(jnp.uint32(0xFFFF0000), (16,))

        zeros16 = jnp.broadcast_to(jnp.float32(0.0), (16,))
        for s in range(NBUF):
            for j in range(NVR):
                buf_v[s, pl.ds(16 * j, 16)] = zeros16

        def gathers(li, s):
            return [
                (q_hbm.at[idx_v.at[li, 0]], buf_v.at[s].at[pl.ds(0, HALF)]),
                (q_hbm.at[idx_v.at[li, 1]], buf_v.at[s].at[pl.ds(PADH, HALF)]),
            ]

        def issue(li, s):
            for src, dst in gathers(li, s):
                pltpu.async_copy(src, dst, sems.at[s])

        def drain(li, s):
            for src, dst in gathers(li, s):
                pltpu.make_async_copy(src, dst, sems.at[s]).wait()

        def reduce(s, v1, v2):
            acc1 = zeros16
            acc2 = zeros16
            for j in range(NVR):
                w = plsc.bitcast(buf_v[s, pl.ds(16 * j, 16)], jnp.uint32)
                acc1 = acc1 + plsc.bitcast(w << 16, jnp.float32)
                acc2 = acc2 + plsc.bitcast(w & himask, jnp.float32)
            s1 = jnp.sum(acc1) + b1s
            s2 = jnp.sum(acc2) + b2s
            sel = lanes == s
            v1 = jnp.where(sel, jnp.broadcast_to(s1, (16,)), v1)
            v2 = jnp.where(sel, jnp.broadcast_to(s2, (16,)), v2)
            return v1, v2

        def chunk_body(ci, _):
            cbase = base + ci * CHUNK
            pltpu.sync_copy(x_hbm.at[pl.ds(cbase, CHUNK)], idx_v)
            for b in range(NBUF):
                issue(b, b)

            def group_body(g, _):
                v1 = zeros16
                v2 = zeros16
                for b in range(NBUF):
                    li = g * NBUF + b
                    drain(li, b)
                    v1, v2 = reduce(b, v1, v2)

                    @pl.when(li + NBUF < CHUNK)
                    def _():
                        issue(li + NBUF, b)
                o1_v[pl.ds(g * NBUF, 16)] = v1
                o2_v[pl.ds(g * NBUF, 16)] = v2
                return 0

            lax.fori_loop(0, NGROUPS, group_body, 0)
            pltpu.sync_copy(o1_v, out1_hbm.at[pl.ds(cbase, CHUNK)])
            pltpu.sync_copy(o2_v, out2_hbm.at[pl.ds(cbase, CHUNK)])
            return 0

        lax.fori_loop(0, NCHUNKS, chunk_body, 0)

    return body(x3, q, bias_vec)


@jax.jit
def kernel(x, table, W1, b1, W2, b2):
    w12 = jnp.concatenate([W1, W2], axis=0) * (1.0 / HIST)
    bias_vec = jnp.concatenate(
        [b1, b2, jnp.zeros((14,), jnp.float32)])
    q = _tc_heads_packed(table, w12)
    x3 = x.astype(jnp.int32).reshape(BATCH, 2, HALF)
    out1, out2 = _sc_gather_reduce(x3, q, bias_vec)
    return (out1, out2)


# speedup vs baseline: 1.4258x; 1.0408x over previous
import functools

import jax
import jax.numpy as jnp
from jax import lax
from jax.experimental import pallas as pl
from jax.experimental.pallas import tpu as pltpu
from jax.experimental.pallas import tpu_sc as plsc

NUM_ROWS = 1_000_000
DIM = 64
BATCH = 16384
HIST = 200
HALF = HIST // 2
PADH = 104
BUF = 2 * PADH
NVR = BUF // 16

NC = 2
NS = 16
NW = NC * NS
BPW = BATCH // NW
CHUNK = 256
NCHUNKS = BPW // CHUNK
NBUF = 16
NGROUPS = CHUNK // NBUF


def _tc_heads_packed(table, w12):
    rb = 32768
    grid = pl.cdiv(NUM_ROWS, rb)

    def body(t_ref, w_ref, o_ref):
        r = lax.dot_general(w_ref[...], t_ref[...], (((1,), (1,)), ((), ())),
                            preferred_element_type=jnp.float32)
        u0 = lax.bitcast_convert_type(r[0], jnp.uint32)
        u1 = lax.bitcast_convert_type(r[1], jnp.uint32)
        r0 = (u0 + jnp.uint32(0x7FFF) + ((u0 >> 16) & jnp.uint32(1))) \
            & jnp.uint32(0xFFFF0000)
        r1 = (u1 + jnp.uint32(0x7FFF) + ((u1 >> 16) & jnp.uint32(1))) \
            & jnp.uint32(0xFFFF0000)
        q = (r0 >> 16) | r1
        o_ref[...] = lax.bitcast_convert_type(q, jnp.float32)

    return pl.pallas_call(
        body,
        grid=(grid,),
        in_specs=[
            pl.BlockSpec((rb, DIM), lambda i: (i, 0)),
            pl.BlockSpec((2, DIM), lambda i: (0, 0)),
        ],
        out_specs=pl.BlockSpec((rb,), lambda i: (i,)),
        out_shape=jax.ShapeDtypeStruct((NUM_ROWS,), jnp.float32),
    )(table, w12)


def _sc_gather_reduce(x3, q, bias_vec):
    mesh = plsc.VectorSubcoreMesh(core_axis_name="c", subcore_axis_name="s",
                                  num_cores=NC, num_subcores=NS)

    @functools.partial(
        pl.kernel,
        out_type=[jax.ShapeDtypeStruct((BATCH,), jnp.float32),
                  jax.ShapeDtypeStruct((BATCH,), jnp.float32)],
        mesh=mesh,
        scratch_types=[
            pltpu.VMEM((CHUNK, 2, HALF), jnp.int32),
            pltpu.VMEM((NBUF, BUF), jnp.float32),
            pltpu.VMEM((CHUNK,), jnp.float32),
            pltpu.VMEM((CHUNK,), jnp.float32),
            pltpu.VMEM((16,), jnp.float32),
            pltpu.SemaphoreType.DMA((NBUF,)),
        ],
        compiler_params=pltpu.CompilerParams(use_tc_tiling_on_sc=False,
                                             needs_layout_passes=False),
    )
    def body(x_hbm, q_hbm, bias_hbm, out1_hbm, out2_hbm,
             idx_v, buf_v, o1_v, o2_v, bias_v, sems):
        wid = lax.axis_index("s") * NC + lax.axis_index("c")
        base = wid * BPW
        pltpu.sync_copy(bias_hbm, bias_v)
        bv = bias_v[...]
        b1s = bv[0]
        b2s = bv[1]
        lanes = lax.iota(jnp.int32, 16)
        himask = jnp.broadcast_to(jnp.uint32(0xFFFF0000), (16,))

        zeros16 = jnp.broadcast_to(jnp.float32(0.0), (16,))
        for s in range(NBUF):
            for j in range(NVR):
                buf_v[s, pl.ds(16 * j, 16)] = zeros16

        def gathers(li, s):
            return [
                (q_hbm.at[idx_v.at[li, 0]], buf_v.at[s].at[pl.ds(0, HALF)]),
                (q_hbm.at[idx_v.at[li, 1]], buf_v.at[s].at[pl.ds(PADH, HALF)]),
            ]

        def issue(li, s):
            for src, dst in gathers(li, s):
                pltpu.async_copy(src, dst, sems.at[s])

        def drain(li, s):
            for src, dst in gathers(li, s):
                pltpu.make_async_copy(src, dst, sems.at[s]).wait()

        def reduce(s, v1, v2):
            acc1 = zeros16
            acc2 = zeros16
            for j in range(NVR):
                w = plsc.bitcast(buf_v[s, pl.ds(16 * j, 16)], jnp.uint32)
                acc1 = acc1 + plsc.bitcast(w << 16, jnp.float32)
                acc2 = acc2 + plsc.bitcast(w & himask, jnp.float32)
            s1 = jnp.sum(acc1) + b1s
            s2 = jnp.sum(acc2) + b2s
            sel = lanes == s
            v1 = jnp.where(sel, jnp.broadcast_to(s1, (16,)), v1)
            v2 = jnp.where(sel, jnp.broadcast_to(s2, (16,)), v2)
            return v1, v2

        def chunk_body(ci, _):
            cbase = base + ci * CHUNK
            pltpu.sync_copy(x_hbm.at[pl.ds(cbase, CHUNK)], idx_v)
            for b in range(NBUF):
                issue(b, b)

            def group_body(g, _):
                v1 = zeros16
                v2 = zeros16
                for b in range(NBUF):
                    li = g * NBUF + b
                    drain(li, b)
                    v1, v2 = reduce(b, v1, v2)

                    @pl.when(li + NBUF < CHUNK)
                    def _():
                        issue(li + NBUF, b)
                o1_v[pl.ds(g * NBUF, 16)] = v1
                o2_v[pl.ds(g * NBUF, 16)] = v2
                return 0

            lax.fori_loop(0, NGROUPS, group_body, 0)
            pltpu.sync_copy(o1_v, out1_hbm.at[pl.ds(cbase, CHUNK)])
            pltpu.sync_copy(o2_v, out2_hbm.at[pl.ds(cbase, CHUNK)])
            return 0

        lax.fori_loop(0, NCHUNKS, chunk_body, 0)

    return body(x3, q, bias_vec)


@jax.jit
def kernel(x, table, W1, b1, W2, b2):
    w12 = jnp.concatenate([W1, W2], axis=0) * (1.0 / HIST)
    bias_vec = jnp.concatenate(
        [b1, b2, jnp.zeros((14,), jnp.float32)])
    q = _tc_heads_packed(table, w12)
    x3 = x.astype(jnp.int32).reshape(BATCH, 2, HALF)
    out1, out2 = _sc_gather_reduce(x3, q, bias_vec)
    return (out1, out2)
